# baseline (speedup 1.0000x reference)
import jax
import jax.numpy as jnp
from jax import lax
from jax.experimental import pallas as pl
from jax.experimental.pallas import tpu as pltpu

_N = 4096
_BN = 512
_NBLK = _N // _BN
_K = 256
_I32 = jnp.int32
_BF = jnp.bfloat16


def _mono_key(f):
    b = lax.bitcast_convert_type(f, _I32)
    return b ^ jnp.where(b < 0, jnp.int32(0x7FFFFFFF), jnp.int32(0))


def _count(mask):
    return jnp.sum(mask.astype(_I32))


def _bit_build(test, t0, bits):
    t = t0
    i = 0
    while i < len(bits):
        if i + 1 < len(bits):
            bh, bl = 1 << bits[i], 1 << bits[i + 1]
            c1 = t | jnp.int32(bh)
            c2 = t | jnp.int32(bl)
            c3 = t | jnp.int32(bh | bl)
            n1, n2, n3 = test(c1), test(c2), test(c3)
            t = jnp.where(n1, jnp.where(n3, c3, c1), jnp.where(n2, c2, t))
            i += 2
        else:
            c1 = t | jnp.int32(1 << bits[i])
            t = jnp.where(test(c1), c1, t)
            i += 1
    return t


def _topk_tail(s, vals_ref, idx_ref):
    key = _mono_key(s)
    flat_iota = (
        lax.broadcasted_iota(_I32, (_NBLK, _BN), 0) * _BN
        + lax.broadcasted_iota(_I32, (_NBLK, _BN), 1)
    )

    cnt0 = _count(key >= 0)
    t0 = jnp.where(cnt0 >= _K, jnp.int32(0), jnp.int32(-(2**31)))
    t = _bit_build(lambda c: _count(key >= c) >= _K,
                   t0, list(range(30, -1, -1)))

    need = _K - _count(key > t)
    eqm = key == t
    cut = _bit_build(lambda c: _count(eqm & (flat_iota < c)) <= need,
                     jnp.int32(0), list(range(12, -1, -1)))

    selb = (key > t) | (eqm & (flat_iota < cut))

    selbf = selb.astype(_BF)
    dn_std = (((1,), (0,)), ((), ()))
    tri_u = (
        lax.broadcasted_iota(_I32, (_BN, _BN), 0)
        <= lax.broadcasted_iota(_I32, (_BN, _BN), 1)
    ).astype(_BF)
    x = lax.dot_general(selbf, tri_u, dn_std,
                        preferred_element_type=jnp.float32)
    row_tot = x[:, _BN - 1 : _BN]
    tri_l = (
        lax.broadcasted_iota(_I32, (_NBLK, _NBLK), 1)
        < lax.broadcasted_iota(_I32, (_NBLK, _NBLK), 0)
    ).astype(jnp.float32)
    row_off = lax.dot_general(tri_l, row_tot, dn_std,
                              preferred_element_type=jnp.float32)
    pos = (x + row_off).astype(_I32) - 1

    posm = jnp.where(selb, pos, jnp.int32(-1))
    posm_t = jnp.transpose(posm)
    sbits = lax.bitcast_convert_type(s, _I32)
    iota_kr = lax.broadcasted_iota(_I32, (1, _K), 1)
    acc = jnp.zeros((6, _K), jnp.float32)
    for r in range(_NBLK):
        p_r = (posm_t[:, r : r + 1] == iota_kr).astype(_BF)
        vb = sbits[r : r + 1, :]
        it = flat_iota[r : r + 1, :]
        payload = jnp.concatenate(
            [((jnp.right_shift(vb, 8 * i) & 255).astype(_BF))
             for i in range(4)]
            + [(it & 255).astype(_BF),
               jnp.right_shift(it, 8).astype(_BF)],
            axis=0,
        )
        acc += lax.dot_general(payload, p_r, dn_std,
                               preferred_element_type=jnp.float32)

    def _reassemble(mat6):
        b = [mat6[i : i + 1, :].astype(_I32) for i in range(6)]
        vbits = b[0] | (b[1] << 8) | (b[2] << 16) | (b[3] << 24)
        vals = lax.bitcast_convert_type(vbits, jnp.float32)
        idxs = b[4] | (b[5] << 8)
        return vals, idxs

    cvals_row, cidx_row = _reassemble(acc)

    ckey_row = _mono_key(cvals_row)
    ckey_col = jnp.transpose(ckey_row)
    cidxi_col = jnp.transpose(cidx_row)
    before = (ckey_col < ckey_row) | (
        (ckey_col == ckey_row) & (cidxi_col > cidx_row)
    )
    crank_col = jnp.sum(before.astype(_I32), axis=1, keepdims=True)
    onehot = (crank_col == iota_kr).astype(_BF)
    payload_c = jnp.concatenate(
        [((jnp.right_shift(lax.bitcast_convert_type(cvals_row, _I32),
                           8 * i) & 255).astype(_BF))
         for i in range(4)]
        + [(cidx_row & 255).astype(_BF),
           jnp.right_shift(cidx_row, 8).astype(_BF)],
        axis=0,
    )
    out6 = lax.dot_general(payload_c, onehot, dn_std,
                           preferred_element_type=jnp.float32)
    out_vals, out_idx = _reassemble(out6)
    vals_ref[...] = out_vals
    idx_ref[...] = out_idx


def _body(x_ref, w_ref, vals_ref, idx_ref, scores_ref):
    j = pl.program_id(0)
    part = jnp.dot(x_ref[...], w_ref[...], preferred_element_type=jnp.float32)
    scores_ref[pl.ds(j, 1), :] = jnp.sum(part, axis=0, keepdims=True)

    @pl.when(j == _NBLK - 1)
    def _():
        _topk_tail(scores_ref[...], vals_ref, idx_ref)


def kernel(in_values, weights, topk):
    del topk
    vals, idxs = pl.pallas_call(
        _body,
        grid=(_NBLK,),
        in_specs=[
            pl.BlockSpec((32, _N), lambda j: (0, 0)),
            pl.BlockSpec((_N, _BN), lambda j: (0, j)),
        ],
        out_specs=[
            pl.BlockSpec((1, _K), lambda j: (0, 0)),
            pl.BlockSpec((1, _K), lambda j: (0, 0)),
        ],
        out_shape=[
            jax.ShapeDtypeStruct((1, _K), jnp.float32),
            jax.ShapeDtypeStruct((1, _K), jnp.int32),
        ],
        scratch_shapes=[pltpu.VMEM((_NBLK, _BN), jnp.float32)],
        compiler_params=pltpu.CompilerParams(
            dimension_semantics=("arbitrary",),
        ),
    )(in_values, weights)
    return vals[0], idxs[0]


# speedup vs baseline: 1.0222x; 1.0222x over previous
import jax
import jax.numpy as jnp
from jax import lax
from jax.experimental import pallas as pl
from jax.experimental.pallas import tpu as pltpu

_N = 4096
_BN = 512
_NBLK = _N // _BN
_K = 256
_I32 = jnp.int32
_BF = jnp.bfloat16


def _mono_key(f):
    b = lax.bitcast_convert_type(f, _I32)
    return b ^ jnp.where(b < 0, jnp.int32(0x7FFFFFFF), jnp.int32(0))


def _count(mask):
    return jnp.sum(mask.astype(_I32))


def _bit_build(test, t0, nbits):
    t = t0
    hi = nbits
    while hi > 0:
        g = 3 if hi >= 3 else hi
        shift = hi - g
        passed = [test(t | jnp.int32(v << shift)).astype(_I32)
                  for v in range(1, 1 << g)]
        vstar = passed[0]
        for p in passed[1:]:
            vstar = vstar + p
        t = t | (vstar << shift)
        hi -= g
    return t


def _topk_tail(s, vals_ref, idx_ref):
    key = _mono_key(s)
    flat_iota = (
        lax.broadcasted_iota(_I32, (_NBLK, _BN), 0) * _BN
        + lax.broadcasted_iota(_I32, (_NBLK, _BN), 1)
    )

    cnt0 = _count(key >= 0)
    t0 = jnp.where(cnt0 >= _K, jnp.int32(0), jnp.int32(-(2**31)))
    t = _bit_build(lambda c: _count(key >= c) >= _K, t0, 31)

    need = _K - _count(key > t)
    eqm = key == t
    cut = _bit_build(lambda c: _count(eqm & (flat_iota < c)) <= need,
                     jnp.int32(0), 13)

    selb = (key > t) | (eqm & (flat_iota < cut))

    selbf = selb.astype(_BF)
    dn_std = (((1,), (0,)), ((), ()))
    tri_u = (
        lax.broadcasted_iota(_I32, (_BN, _BN), 0)
        <= lax.broadcasted_iota(_I32, (_BN, _BN), 1)
    ).astype(_BF)
    x = lax.dot_general(selbf, tri_u, dn_std,
                        preferred_element_type=jnp.float32)
    row_tot = x[:, _BN - 1 : _BN]
    tri_l = (
        lax.broadcasted_iota(_I32, (_NBLK, _NBLK), 1)
        < lax.broadcasted_iota(_I32, (_NBLK, _NBLK), 0)
    ).astype(jnp.float32)
    row_off = lax.dot_general(tri_l, row_tot, dn_std,
                              preferred_element_type=jnp.float32)
    pos = (x + row_off).astype(_I32) - 1

    posm = jnp.where(selb, pos, jnp.int32(-1))
    posm_t = jnp.transpose(posm).astype(_BF)
    sbits = lax.bitcast_convert_type(s, _I32)
    iota_kr = lax.broadcasted_iota(_I32, (1, _K), 1)
    iota_kb = iota_kr.astype(_BF)
    acc = jnp.zeros((6, _K), jnp.float32)
    for r in range(_NBLK):
        p_r = (posm_t[:, r : r + 1] == iota_kb).astype(_BF)
        vb = sbits[r : r + 1, :]
        it = flat_iota[r : r + 1, :]
        payload = jnp.concatenate(
            [((jnp.right_shift(vb, 8 * i) & 255).astype(_BF))
             for i in range(4)]
            + [(it & 255).astype(_BF),
               jnp.right_shift(it, 8).astype(_BF)],
            axis=0,
        )
        acc += lax.dot_general(payload, p_r, dn_std,
                               preferred_element_type=jnp.float32)

    def _reassemble(mat6):
        b = [mat6[i : i + 1, :].astype(_I32) for i in range(6)]
        vbits = b[0] | (b[1] << 8) | (b[2] << 16) | (b[3] << 24)
        vals = lax.bitcast_convert_type(vbits, jnp.float32)
        idxs = b[4] | (b[5] << 8)
        return vals, idxs

    cvals_row, cidx_row = _reassemble(acc)

    ckey_row = _mono_key(cvals_row)
    ckey_col = jnp.transpose(ckey_row)
    cidxi_col = jnp.transpose(cidx_row)
    before = (ckey_col < ckey_row) | (
        (ckey_col == ckey_row) & (cidxi_col > cidx_row)
    )
    crank_col = jnp.sum(before.astype(_I32), axis=1, keepdims=True)
    onehot = (crank_col == iota_kr).astype(_BF)
    payload_c = jnp.concatenate(
        [((jnp.right_shift(lax.bitcast_convert_type(cvals_row, _I32),
                           8 * i) & 255).astype(_BF))
         for i in range(4)]
        + [(cidx_row & 255).astype(_BF),
           jnp.right_shift(cidx_row, 8).astype(_BF)],
        axis=0,
    )
    out6 = lax.dot_general(payload_c, onehot, dn_std,
                           preferred_element_type=jnp.float32)
    out_vals, out_idx = _reassemble(out6)
    vals_ref[...] = out_vals
    idx_ref[...] = out_idx


def _body(x_ref, w_ref, vals_ref, idx_ref, scores_ref):
    j = pl.program_id(0)
    part = jnp.dot(x_ref[...], w_ref[...], preferred_element_type=jnp.float32)
    scores_ref[pl.ds(j, 1), :] = jnp.sum(part, axis=0, keepdims=True)

    @pl.when(j == _NBLK - 1)
    def _():
        _topk_tail(scores_ref[...], vals_ref, idx_ref)


def kernel(in_values, weights, topk):
    del topk
    vals, idxs = pl.pallas_call(
        _body,
        grid=(_NBLK,),
        in_specs=[
            pl.BlockSpec((32, _N), lambda j: (0, 0)),
            pl.BlockSpec((_N, _BN), lambda j: (0, j)),
        ],
        out_specs=[
            pl.BlockSpec((1, _K), lambda j: (0, 0)),
            pl.BlockSpec((1, _K), lambda j: (0, 0)),
        ],
        out_shape=[
            jax.ShapeDtypeStruct((1, _K), jnp.float32),
            jax.ShapeDtypeStruct((1, _K), jnp.int32),
        ],
        scratch_shapes=[pltpu.VMEM((_NBLK, _BN), jnp.float32)],
        compiler_params=pltpu.CompilerParams(
            dimension_semantics=("arbitrary",),
        ),
    )(in_values, weights)
    return vals[0], idxs[0]
